# baseline (speedup 1.0000x reference)
import jax
import jax.numpy as jnp
from jax import lax
from jax.experimental import pallas as pl
from jax.experimental.pallas import tpu as pltpu
from jax.experimental.pallas import tpu_sc as plsc

N = 10000
E = 320000
D = 128
QD = 32
NQ = D // QD
EPT = E // 16
CHUNK = 80
NCH = EPT // CHUNK
NB = 2
DW = 48
ROWS_PT = N // 16
ZROWS = 125

_f32 = jnp.float32
_i32 = jnp.int32


def _sc_stage_edges(eidx_hbm, sid, src2, dst2):
    pltpu.sync_copy(eidx_hbm.at[0, sid], src2)
    pltpu.sync_copy(eidx_hbm.at[1, sid], dst2)


def _sc_one_head(zq_hbm, s_hbm, d_hbm, out_hbm, cid, sid,
                 src2, dst2, s_v, d_v, ex2, zbuf, rin, rout, h_acc,
                 sem_g, sem_s):
    pltpu.sync_copy(s_hbm, s_v)
    pltpu.sync_copy(d_hbm, d_v)

    def ex_body(c, carry):
        for g in range(CHUNK // 16):
            si = src2[c, pl.ds(g * 16, 16)]
            di = dst2[c, pl.ds(g * 16, 16)]
            e = plsc.load_gather(s_v, [si]) + plsc.load_gather(d_v, [di])
            e = jnp.where(e >= 0.0, e, 0.2 * e)
            ex2[c, pl.ds(g * 16, 16)] = jnp.exp(e)
        return carry
    lax.fori_loop(0, NCH, ex_body, 0)

    for p in range(NQ // 2):
        qi = 2 * p + cid

        def g_desc(b, c):
            return pltpu.make_async_copy(zq_hbm.at[qi].at[src2.at[c]],
                                         rin.at[b], sem_g.at[b])

        def s_desc(b, c):
            return pltpu.make_async_copy(rout.at[b], h_acc.at[dst2.at[c]],
                                         sem_s.at[b])

        for k in range(ROWS_PT // ZROWS):
            pltpu.sync_copy(
                zbuf, h_acc.at[pl.ds(sid * ROWS_PT + k * ZROWS, ZROWS)])
        plsc.subcore_barrier()

        for b in range(NB):
            g_desc(b, b).start()

        def chunk_body(c, carry):
            b = c % NB
            g_desc(b, c).wait()

            @pl.when(c >= NB)
            def _():
                s_desc(b, c).wait()
            rin_b = rin.at[b]
            rout_b = rout.at[b]
            for g in range(CHUNK // 16):
                exv = ex2[c, pl.ds(g * 16, 16)]
                for l in range(16):
                    j = g * 16 + l
                    es = jnp.broadcast_to(exv[l], (16,))
                    for r in range(QD // 16):
                        rout_b[j, pl.ds(r * 16, 16)] = (
                            rin_b[j, pl.ds(r * 16, 16)] * es)
                    rout_b[j, pl.ds(QD, 16)] = es
            pltpu.async_copy(rout.at[b], h_acc.at[dst2.at[c]],
                             sem_s.at[b], add=True)

            @pl.when(c + NB < NCH)
            def _():
                g_desc(b, c + NB).start()
            return carry
        lax.fori_loop(0, NCH, chunk_body, 0)
        for b in range(NB):
            s_desc(b, NCH - 1).wait()

        plsc.subcore_barrier()
        pltpu.sync_copy(h_acc.at[pl.ds(sid * ROWS_PT, ROWS_PT)],
                        out_hbm.at[qi, pl.ds(sid * ROWS_PT, ROWS_PT)])


def _zero_zbuf(zbuf):
    zv = jnp.zeros((16,), _f32)

    def zero_body(r, carry):
        for q in range(DW // 16):
            zbuf[r, pl.ds(q * 16, 16)] = zv
        return carry
    lax.fori_loop(0, ZROWS, zero_body, 0)


def _sc_layer1_main(z0_hbm, z1_hbm, s0_hbm, d0_hbm, s1_hbm, d1_hbm, eidx_hbm,
                    out_hbm,
                    src2, dst2, s_v, d_v, ex2, zbuf, rin, rout, h_acc,
                    sem_g, sem_s):
    cid = lax.axis_index("c")
    sid = lax.axis_index("s")
    _sc_stage_edges(eidx_hbm, sid, src2, dst2)
    _zero_zbuf(zbuf)
    for hidx, (zh, sh, dh) in enumerate(((z0_hbm, s0_hbm, d0_hbm),
                                         (z1_hbm, s1_hbm, d1_hbm))):
        if hidx:
            plsc.subcore_barrier()
        _sc_one_head(zh, sh, dh, out_hbm.at[hidx], cid, sid,
                     src2, dst2, s_v, d_v, ex2, zbuf, rin, rout, h_acc,
                     sem_g, sem_s)


def _sc_layer2_main(z_hbm, s_hbm, d_hbm, eidx_hbm, out_hbm,
                    src2, dst2, s_v, d_v, ex2, zbuf, rin, rout, h_acc,
                    sem_g, sem_s):
    cid = lax.axis_index("c")
    sid = lax.axis_index("s")
    _sc_stage_edges(eidx_hbm, sid, src2, dst2)
    _zero_zbuf(zbuf)
    _sc_one_head(z_hbm, s_hbm, d_hbm, out_hbm, cid, sid,
                 src2, dst2, s_v, d_v, ex2, zbuf, rin, rout, h_acc,
                 sem_g, sem_s)


_SC_SCRATCH = [
    pltpu.VMEM((NCH, CHUNK), _i32),
    pltpu.VMEM((NCH, CHUNK), _i32),
    pltpu.VMEM((N,), _f32),
    pltpu.VMEM((N,), _f32),
    pltpu.VMEM((NCH, CHUNK), _f32),
    pltpu.VMEM((ZROWS, DW), _f32),
    pltpu.VMEM((NB, CHUNK, QD), _f32),
    pltpu.VMEM((NB, CHUNK, DW), _f32),
    pltpu.VMEM_SHARED((N, DW), _f32),
    pltpu.SemaphoreType.DMA((NB,)),
    pltpu.SemaphoreType.DMA((NB,)),
]

_SC_PARAMS = pltpu.CompilerParams(use_tc_tiling_on_sc=False,
                                  needs_layout_passes=False)

_MESH = plsc.VectorSubcoreMesh(core_axis_name="c", subcore_axis_name="s")

_sc_layer1 = pl.kernel(
    _sc_layer1_main,
    out_type=jax.ShapeDtypeStruct((2, NQ, N, DW), _f32),
    mesh=_MESH,
    scratch_types=_SC_SCRATCH,
    compiler_params=_SC_PARAMS,
)

_sc_layer2 = pl.kernel(
    _sc_layer2_main,
    out_type=jax.ShapeDtypeStruct((NQ, N, DW), _f32),
    mesh=_MESH,
    scratch_types=_SC_SCRATCH,
    compiler_params=_SC_PARAMS,
)


_BM = 1000


def _tc1_body(x_ref, wc_ref, z0_ref, z1_ref, sd_ref):
    acc = jnp.dot(x_ref[...], wc_ref[...], preferred_element_type=_f32)
    for q in range(NQ):
        z0_ref[q] = acc[:, QD * q:QD * (q + 1)]
        z1_ref[q] = acc[:, D + QD * q:D + QD * (q + 1)]
    sd_ref[...] = acc[:, 2 * D:2 * D + 8]


def _tc1(x, wc):
    return pl.pallas_call(
        _tc1_body,
        grid=(N // _BM,),
        in_specs=[
            pl.BlockSpec((_BM, D), lambda i: (i, 0)),
            pl.BlockSpec((D, 2 * D + 8), lambda i: (0, 0)),
        ],
        out_specs=[
            pl.BlockSpec((NQ, _BM, QD), lambda i: (0, i, 0)),
            pl.BlockSpec((NQ, _BM, QD), lambda i: (0, i, 0)),
            pl.BlockSpec((_BM, 8), lambda i: (i, 0)),
        ],
        out_shape=[
            jax.ShapeDtypeStruct((NQ, N, QD), _f32),
            jax.ShapeDtypeStruct((NQ, N, QD), _f32),
            jax.ShapeDtypeStruct((N, 8), _f32),
        ],
    )(x, wc)


def _gat_merge(p):
    den = p[0, :, QD:QD + 1]
    den = jnp.where(den == 0.0, 1.0, den)
    return jnp.concatenate([p[q, :, :QD] for q in range(NQ)], axis=1) / den


def _tc2_body(ph_ref, wc2_ref, z2_ref, sd2_ref):
    h = jnp.concatenate([_gat_merge(ph_ref[0]), _gat_merge(ph_ref[1])],
                        axis=1)
    acc = jnp.dot(h, wc2_ref[...], preferred_element_type=_f32)
    for q in range(NQ):
        z2_ref[q] = acc[:, QD * q:QD * (q + 1)]
    sd2_ref[...] = acc[:, D:D + 8]


def _tc2(ph, wc2):
    return pl.pallas_call(
        _tc2_body,
        grid=(N // _BM,),
        in_specs=[
            pl.BlockSpec((2, NQ, _BM, DW), lambda i: (0, 0, i, 0)),
            pl.BlockSpec((2 * D, D + 8), lambda i: (0, 0)),
        ],
        out_specs=[
            pl.BlockSpec((NQ, _BM, QD), lambda i: (0, i, 0)),
            pl.BlockSpec((_BM, 8), lambda i: (i, 0)),
        ],
        out_shape=[
            jax.ShapeDtypeStruct((NQ, N, QD), _f32),
            jax.ShapeDtypeStruct((N, 8), _f32),
        ],
    )(ph, wc2)


def _tc3_body(q_ref, out_ref):
    out_ref[...] = _gat_merge(q_ref[...])


def _tc3(q):
    return pl.pallas_call(
        _tc3_body,
        grid=(N // _BM,),
        in_specs=[pl.BlockSpec((NQ, _BM, DW), lambda i: (0, i, 0))],
        out_specs=pl.BlockSpec((_BM, D), lambda i: (i, 0)),
        out_shape=jax.ShapeDtypeStruct((N, D), _f32),
    )(q)


def kernel(features, edge_index, W1_0, a1_0, W1_1, a1_1, W2_0, a2_0):
    ws0 = W1_0 @ a1_0[:D, 0]
    wd0 = W1_0 @ a1_0[D:, 0]
    ws1 = W1_1 @ a1_1[:D, 0]
    wd1 = W1_1 @ a1_1[D:, 0]
    zpad = jnp.zeros_like(ws0)
    sdw1 = jnp.stack([ws0, wd0, ws1, wd1, zpad, zpad, zpad, zpad], axis=1)
    wc1 = jnp.concatenate([W1_0, W1_1, sdw1], axis=1)

    z0, z1, sd = _tc1(features, wc1)
    eidx = edge_index.reshape(2, 16, NCH, CHUNK)

    ph = _sc_layer1(z0, z1, sd[:, 0], sd[:, 1], sd[:, 2], sd[:, 3], eidx)

    ws2 = W2_0 @ a2_0[:D, 0]
    wd2 = W2_0 @ a2_0[D:, 0]
    zpad2 = jnp.zeros_like(ws2)
    sdw2 = jnp.stack([ws2, wd2] + [zpad2] * 6, axis=1)
    wc2 = jnp.concatenate([W2_0, sdw2], axis=1)

    z2, sd2 = _tc2(ph, wc2)
    q = _sc_layer2(z2, sd2[:, 0], sd2[:, 1], eidx)
    return _tc3(q)


# speedup vs baseline: 1.8536x; 1.8536x over previous
import jax
import jax.numpy as jnp
from jax import lax
from jax.experimental import pallas as pl
from jax.experimental.pallas import tpu as pltpu
from jax.experimental.pallas import tpu_sc as plsc

N = 10000
E = 320000
D = 128
QD = 32
NQ = D // QD
EPT = E // 16
CHUNK = 160
NCH = EPT // CHUNK
NB = 2
DW = 48
ROWS_PT = N // 16
ZROWS = 125

_f32 = jnp.float32
_i32 = jnp.int32


def _sc_stage_edges(eidx_hbm, sid, src2, dst2):
    pltpu.sync_copy(eidx_hbm.at[0, sid], src2)
    pltpu.sync_copy(eidx_hbm.at[1, sid], dst2)


def _sc_one_head(zq_hbm, s_hbm, d_hbm, out_hbm, cid, sid,
                 src2, dst2, s_v, d_v, zbuf, rin, rout, h_acc,
                 sem_g, sem_s):
    pltpu.sync_copy(s_hbm, s_v)
    pltpu.sync_copy(d_hbm, d_v)

    for p in range(NQ // 2):
        qi = 2 * p + cid

        def g_desc(b, c):
            return pltpu.make_async_copy(zq_hbm.at[qi].at[src2.at[c]],
                                         rin.at[b], sem_g.at[b])

        def s_desc(b, c):
            return pltpu.make_async_copy(rout.at[b], h_acc.at[dst2.at[c]],
                                         sem_s.at[b])

        for k in range(ROWS_PT // ZROWS):
            pltpu.sync_copy(
                zbuf, h_acc.at[pl.ds(sid * ROWS_PT + k * ZROWS, ZROWS)])
        plsc.subcore_barrier()

        def compute_scale(b, c):
            for g in range(CHUNK // 16):
                si = src2[c, pl.ds(g * 16, 16)]
                di = dst2[c, pl.ds(g * 16, 16)]
                e = plsc.load_gather(s_v, [si]) + plsc.load_gather(d_v, [di])
                e = jnp.where(e >= 0.0, e, 0.2 * e)
                exv = jnp.exp(e)
                for l in range(16):
                    j = g * 16 + l
                    es = jnp.broadcast_to(exv[l], (16,))
                    for r in range(QD // 16):
                        rout[b, j, pl.ds(r * 16, 16)] = (
                            rin[b, j, pl.ds(r * 16, 16)] * es)
                    rout[b, j, pl.ds(QD, 16)] = es
            pltpu.async_copy(rout.at[b], h_acc.at[dst2.at[c]],
                             sem_s.at[b], add=True)

        for b in range(2):
            g_desc(b, b).start()

        def chunk_body(cc, carry):
            for b in range(2):
                c = 2 * cc + b
                g_desc(b, c).wait()

                @pl.when(cc > 0)
                def _():
                    s_desc(b, c).wait()
                compute_scale(b, c)
                g_desc(b, jnp.minimum(c + 2, NCH - 1)).start()
            return carry
        lax.fori_loop(0, NCH // 2, chunk_body, 0)
        if NCH % 2:
            ce = NCH - 1
            g_desc(0, ce).wait()
            s_desc(0, ce).wait()
            compute_scale(0, ce)
            g_desc(1, ce).wait()
        else:
            for b in range(2):
                g_desc(b, NCH - 1).wait()
        for b in range(2):
            s_desc(b, NCH - 1).wait()

        plsc.subcore_barrier()
        pltpu.sync_copy(h_acc.at[pl.ds(sid * ROWS_PT, ROWS_PT)],
                        out_hbm.at[qi, pl.ds(sid * ROWS_PT, ROWS_PT)])


def _zero_zbuf(zbuf):
    zv = jnp.zeros((16,), _f32)

    def zero_body(r, carry):
        for q in range(DW // 16):
            zbuf[r, pl.ds(q * 16, 16)] = zv
        return carry
    lax.fori_loop(0, ZROWS, zero_body, 0)


def _sc_layer1_main(z0_hbm, z1_hbm, s0_hbm, d0_hbm, s1_hbm, d1_hbm, eidx_hbm,
                    out_hbm,
                    src2, dst2, s_v, d_v, zbuf, rin, rout, h_acc,
                    sem_g, sem_s):
    cid = lax.axis_index("c")
    sid = lax.axis_index("s")
    _sc_stage_edges(eidx_hbm, sid, src2, dst2)
    _zero_zbuf(zbuf)
    for hidx, (zh, sh, dh) in enumerate(((z0_hbm, s0_hbm, d0_hbm),
                                         (z1_hbm, s1_hbm, d1_hbm))):
        if hidx:
            plsc.subcore_barrier()
        _sc_one_head(zh, sh, dh, out_hbm.at[hidx], cid, sid,
                     src2, dst2, s_v, d_v, zbuf, rin, rout, h_acc,
                     sem_g, sem_s)


def _sc_layer2_main(z_hbm, s_hbm, d_hbm, eidx_hbm, out_hbm,
                    src2, dst2, s_v, d_v, zbuf, rin, rout, h_acc,
                    sem_g, sem_s):
    cid = lax.axis_index("c")
    sid = lax.axis_index("s")
    _sc_stage_edges(eidx_hbm, sid, src2, dst2)
    _zero_zbuf(zbuf)
    _sc_one_head(z_hbm, s_hbm, d_hbm, out_hbm, cid, sid,
                 src2, dst2, s_v, d_v, zbuf, rin, rout, h_acc,
                 sem_g, sem_s)


_SC_SCRATCH = [
    pltpu.VMEM((NCH, CHUNK), _i32),
    pltpu.VMEM((NCH, CHUNK), _i32),
    pltpu.VMEM((N,), _f32),
    pltpu.VMEM((N,), _f32),
    pltpu.VMEM((ZROWS, DW), _f32),
    pltpu.VMEM((NB, CHUNK, QD), _f32),
    pltpu.VMEM((NB, CHUNK, DW), _f32),
    pltpu.VMEM_SHARED((N, DW), _f32),
    pltpu.SemaphoreType.DMA((NB,)),
    pltpu.SemaphoreType.DMA((NB,)),
]

_SC_PARAMS = pltpu.CompilerParams(use_tc_tiling_on_sc=False,
                                  needs_layout_passes=False)

_MESH = plsc.VectorSubcoreMesh(core_axis_name="c", subcore_axis_name="s")

_sc_layer1 = pl.kernel(
    _sc_layer1_main,
    out_type=jax.ShapeDtypeStruct((2, NQ, N, DW), _f32),
    mesh=_MESH,
    scratch_types=_SC_SCRATCH,
    compiler_params=_SC_PARAMS,
)

_sc_layer2 = pl.kernel(
    _sc_layer2_main,
    out_type=jax.ShapeDtypeStruct((NQ, N, DW), _f32),
    mesh=_MESH,
    scratch_types=_SC_SCRATCH,
    compiler_params=_SC_PARAMS,
)


_BM = 1000


def _tc1_body(x_ref, wc_ref, z0_ref, z1_ref, sd_ref):
    acc = jnp.dot(x_ref[...], wc_ref[...], preferred_element_type=_f32)
    for q in range(NQ):
        z0_ref[q] = acc[:, QD * q:QD * (q + 1)]
        z1_ref[q] = acc[:, D + QD * q:D + QD * (q + 1)]
    sd_ref[...] = acc[:, 2 * D:2 * D + 8]


def _tc1(x, wc):
    return pl.pallas_call(
        _tc1_body,
        grid=(N // _BM,),
        in_specs=[
            pl.BlockSpec((_BM, D), lambda i: (i, 0)),
            pl.BlockSpec((D, 2 * D + 8), lambda i: (0, 0)),
        ],
        out_specs=[
            pl.BlockSpec((NQ, _BM, QD), lambda i: (0, i, 0)),
            pl.BlockSpec((NQ, _BM, QD), lambda i: (0, i, 0)),
            pl.BlockSpec((_BM, 8), lambda i: (i, 0)),
        ],
        out_shape=[
            jax.ShapeDtypeStruct((NQ, N, QD), _f32),
            jax.ShapeDtypeStruct((NQ, N, QD), _f32),
            jax.ShapeDtypeStruct((N, 8), _f32),
        ],
    )(x, wc)


def _gat_merge(p):
    den = p[0, :, QD:QD + 1]
    den = jnp.where(den == 0.0, 1.0, den)
    return jnp.concatenate([p[q, :, :QD] for q in range(NQ)], axis=1) / den


def _tc2_body(ph_ref, wc2_ref, z2_ref, sd2_ref):
    h = jnp.concatenate([_gat_merge(ph_ref[0]), _gat_merge(ph_ref[1])],
                        axis=1)
    acc = jnp.dot(h, wc2_ref[...], preferred_element_type=_f32)
    for q in range(NQ):
        z2_ref[q] = acc[:, QD * q:QD * (q + 1)]
    sd2_ref[...] = acc[:, D:D + 8]


def _tc2(ph, wc2):
    return pl.pallas_call(
        _tc2_body,
        grid=(N // _BM,),
        in_specs=[
            pl.BlockSpec((2, NQ, _BM, DW), lambda i: (0, 0, i, 0)),
            pl.BlockSpec((2 * D, D + 8), lambda i: (0, 0)),
        ],
        out_specs=[
            pl.BlockSpec((NQ, _BM, QD), lambda i: (0, i, 0)),
            pl.BlockSpec((_BM, 8), lambda i: (i, 0)),
        ],
        out_shape=[
            jax.ShapeDtypeStruct((NQ, N, QD), _f32),
            jax.ShapeDtypeStruct((N, 8), _f32),
        ],
    )(ph, wc2)


def _tc3_body(q_ref, out_ref):
    out_ref[...] = _gat_merge(q_ref[...])


def _tc3(q):
    return pl.pallas_call(
        _tc3_body,
        grid=(N // _BM,),
        in_specs=[pl.BlockSpec((NQ, _BM, DW), lambda i: (0, i, 0))],
        out_specs=pl.BlockSpec((_BM, D), lambda i: (i, 0)),
        out_shape=jax.ShapeDtypeStruct((N, D), _f32),
    )(q)


def kernel(features, edge_index, W1_0, a1_0, W1_1, a1_1, W2_0, a2_0):
    ws0 = W1_0 @ a1_0[:D, 0]
    wd0 = W1_0 @ a1_0[D:, 0]
    ws1 = W1_1 @ a1_1[:D, 0]
    wd1 = W1_1 @ a1_1[D:, 0]
    zpad = jnp.zeros_like(ws0)
    sdw1 = jnp.stack([ws0, wd0, ws1, wd1, zpad, zpad, zpad, zpad], axis=1)
    wc1 = jnp.concatenate([W1_0, W1_1, sdw1], axis=1)

    z0, z1, sd = _tc1(features, wc1)
    eidx = edge_index.reshape(2, 16, NCH, CHUNK)

    ph = _sc_layer1(z0, z1, sd[:, 0], sd[:, 1], sd[:, 2], sd[:, 3], eidx)

    ws2 = W2_0 @ a2_0[:D, 0]
    wd2 = W2_0 @ a2_0[D:, 0]
    zpad2 = jnp.zeros_like(ws2)
    sdw2 = jnp.stack([ws2, wd2] + [zpad2] * 6, axis=1)
    wc2 = jnp.concatenate([W2_0, sdw2], axis=1)

    z2, sd2 = _tc2(ph, wc2)
    q = _sc_layer2(z2, sd2[:, 0], sd2[:, 1], eidx)
    return _tc3(q)


# speedup vs baseline: 1.8803x; 1.0144x over previous
import jax
import jax.numpy as jnp
from jax import lax
from jax.experimental import pallas as pl
from jax.experimental.pallas import tpu as pltpu
from jax.experimental.pallas import tpu_sc as plsc

N = 10000
E = 320000
D = 128
QD = 32
NQ = D // QD
EPT = E // 16
CHUNK = 160
NCH = EPT // CHUNK
NB = 2
DW = 48
ROWS_PT = N // 16
ZROWS = 125

_f32 = jnp.float32
_i32 = jnp.int32
_bf16 = jnp.bfloat16


def _sc_stage_edges(eidx_hbm, sid, src2, dst2):
    pltpu.sync_copy(eidx_hbm.at[0, sid], src2)
    pltpu.sync_copy(eidx_hbm.at[1, sid], dst2)


def _sc_one_head(zq_hbm, s_hbm, d_hbm, out_hbm, cid, sid,
                 src2, dst2, s_v, d_v, zbuf, rin, rout, h_acc,
                 sem_g, sem_s):
    pltpu.sync_copy(s_hbm, s_v)
    pltpu.sync_copy(d_hbm, d_v)

    for p in range(NQ // 2):
        qi = 2 * p + cid

        def g_desc(b, c):
            return pltpu.make_async_copy(zq_hbm.at[qi].at[src2.at[c]],
                                         rin.at[b], sem_g.at[b])

        def s_desc(b, c):
            return pltpu.make_async_copy(rout.at[b], h_acc.at[dst2.at[c]],
                                         sem_s.at[b])

        for k in range(ROWS_PT // ZROWS):
            pltpu.sync_copy(
                zbuf, h_acc.at[pl.ds(sid * ROWS_PT + k * ZROWS, ZROWS)])
        plsc.subcore_barrier()

        def compute_scale(b, c):
            for g in range(CHUNK // 16):
                si = src2[c, pl.ds(g * 16, 16)]
                di = dst2[c, pl.ds(g * 16, 16)]
                e = plsc.load_gather(s_v, [si]) + plsc.load_gather(d_v, [di])
                e = jnp.where(e >= 0.0, e, 0.2 * e)
                exv = jnp.exp(e)
                for l in range(16):
                    j = g * 16 + l
                    es = jnp.broadcast_to(exv[l], (16,))
                    za, zb = plsc.unpack(
                        rin[b, j, pl.ds(0, QD)],
                        format=plsc.PackFormat.INTERLEAVED,
                        preferred_element_type=_f32)
                    rout[b, j, pl.ds(0, 16)] = za * es
                    rout[b, j, pl.ds(16, 16)] = zb * es
                    rout[b, j, pl.ds(QD, 16)] = es
            pltpu.async_copy(rout.at[b], h_acc.at[dst2.at[c]],
                             sem_s.at[b], add=True)

        for b in range(2):
            g_desc(b, b).start()

        def chunk_body(cc, carry):
            for b in range(2):
                c = 2 * cc + b
                g_desc(b, c).wait()

                @pl.when(cc > 0)
                def _():
                    s_desc(b, c).wait()
                compute_scale(b, c)
                g_desc(b, jnp.minimum(c + 2, NCH - 1)).start()
            return carry
        lax.fori_loop(0, NCH // 2, chunk_body, 0)
        if NCH % 2:
            ce = NCH - 1
            g_desc(0, ce).wait()
            s_desc(0, ce).wait()
            compute_scale(0, ce)
            g_desc(1, ce).wait()
        else:
            for b in range(2):
                g_desc(b, NCH - 1).wait()
        for b in range(2):
            s_desc(b, NCH - 1).wait()

        plsc.subcore_barrier()
        pltpu.sync_copy(h_acc.at[pl.ds(sid * ROWS_PT, ROWS_PT)],
                        out_hbm.at[qi, pl.ds(sid * ROWS_PT, ROWS_PT)])


def _zero_zbuf(zbuf):
    zv = jnp.zeros((16,), _f32)

    def zero_body(r, carry):
        for q in range(DW // 16):
            zbuf[r, pl.ds(q * 16, 16)] = zv
        return carry
    lax.fori_loop(0, ZROWS, zero_body, 0)


def _sc_layer1_main(z0_hbm, z1_hbm, s0_hbm, d0_hbm, s1_hbm, d1_hbm, eidx_hbm,
                    out_hbm,
                    src2, dst2, s_v, d_v, zbuf, rin, rout, h_acc,
                    sem_g, sem_s):
    cid = lax.axis_index("c")
    sid = lax.axis_index("s")
    _sc_stage_edges(eidx_hbm, sid, src2, dst2)
    _zero_zbuf(zbuf)
    for hidx, (zh, sh, dh) in enumerate(((z0_hbm, s0_hbm, d0_hbm),
                                         (z1_hbm, s1_hbm, d1_hbm))):
        if hidx:
            plsc.subcore_barrier()
        _sc_one_head(zh, sh, dh, out_hbm.at[hidx], cid, sid,
                     src2, dst2, s_v, d_v, zbuf, rin, rout, h_acc,
                     sem_g, sem_s)


def _sc_layer2_main(z_hbm, s_hbm, d_hbm, eidx_hbm, out_hbm,
                    src2, dst2, s_v, d_v, zbuf, rin, rout, h_acc,
                    sem_g, sem_s):
    cid = lax.axis_index("c")
    sid = lax.axis_index("s")
    _sc_stage_edges(eidx_hbm, sid, src2, dst2)
    _zero_zbuf(zbuf)
    _sc_one_head(z_hbm, s_hbm, d_hbm, out_hbm, cid, sid,
                 src2, dst2, s_v, d_v, zbuf, rin, rout, h_acc,
                 sem_g, sem_s)


_SC_SCRATCH = [
    pltpu.VMEM((NCH, CHUNK), _i32),
    pltpu.VMEM((NCH, CHUNK), _i32),
    pltpu.VMEM((N,), _f32),
    pltpu.VMEM((N,), _f32),
    pltpu.VMEM((ZROWS, DW), _f32),
    pltpu.VMEM((NB, CHUNK, QD), _bf16),
    pltpu.VMEM((NB, CHUNK, DW), _f32),
    pltpu.VMEM_SHARED((N, DW), _f32),
    pltpu.SemaphoreType.DMA((NB,)),
    pltpu.SemaphoreType.DMA((NB,)),
]

_SC_PARAMS = pltpu.CompilerParams(use_tc_tiling_on_sc=False,
                                  needs_layout_passes=False)

_MESH = plsc.VectorSubcoreMesh(core_axis_name="c", subcore_axis_name="s")

_sc_layer1 = pl.kernel(
    _sc_layer1_main,
    out_type=jax.ShapeDtypeStruct((2, NQ, N, DW), _f32),
    mesh=_MESH,
    scratch_types=_SC_SCRATCH,
    compiler_params=_SC_PARAMS,
)

_sc_layer2 = pl.kernel(
    _sc_layer2_main,
    out_type=jax.ShapeDtypeStruct((NQ, N, DW), _f32),
    mesh=_MESH,
    scratch_types=_SC_SCRATCH,
    compiler_params=_SC_PARAMS,
)


_BM = 1000


def _tc1_body(x_ref, wc_ref, z0_ref, z1_ref, sd_ref):
    acc = jnp.dot(x_ref[...], wc_ref[...], preferred_element_type=_f32)
    for q in range(NQ):
        z0_ref[q] = acc[:, QD * q:QD * (q + 1)].astype(_bf16)
        z1_ref[q] = acc[:, D + QD * q:D + QD * (q + 1)].astype(_bf16)
    sd_ref[...] = acc[:, 2 * D:2 * D + 8]


def _tc1(x, wc):
    return pl.pallas_call(
        _tc1_body,
        grid=(N // _BM,),
        in_specs=[
            pl.BlockSpec((_BM, D), lambda i: (i, 0)),
            pl.BlockSpec((D, 2 * D + 8), lambda i: (0, 0)),
        ],
        out_specs=[
            pl.BlockSpec((NQ, _BM, QD), lambda i: (0, i, 0)),
            pl.BlockSpec((NQ, _BM, QD), lambda i: (0, i, 0)),
            pl.BlockSpec((_BM, 8), lambda i: (i, 0)),
        ],
        out_shape=[
            jax.ShapeDtypeStruct((NQ, N, QD), _bf16),
            jax.ShapeDtypeStruct((NQ, N, QD), _bf16),
            jax.ShapeDtypeStruct((N, 8), _f32),
        ],
    )(x, wc)


def _gat_merge(p):
    den = p[0, :, QD:QD + 1]
    den = jnp.where(den == 0.0, 1.0, den)
    return jnp.concatenate([p[q, :, :QD] for q in range(NQ)], axis=1) / den


def _tc2_body(ph_ref, wc2_ref, z2_ref, sd2_ref):
    h = jnp.concatenate([_gat_merge(ph_ref[0]), _gat_merge(ph_ref[1])],
                        axis=1)
    acc = jnp.dot(h, wc2_ref[...], preferred_element_type=_f32)
    for q in range(NQ):
        z2_ref[q] = acc[:, QD * q:QD * (q + 1)].astype(_bf16)
    sd2_ref[...] = acc[:, D:D + 8]


def _tc2(ph, wc2):
    return pl.pallas_call(
        _tc2_body,
        grid=(N // _BM,),
        in_specs=[
            pl.BlockSpec((2, NQ, _BM, DW), lambda i: (0, 0, i, 0)),
            pl.BlockSpec((2 * D, D + 8), lambda i: (0, 0)),
        ],
        out_specs=[
            pl.BlockSpec((NQ, _BM, QD), lambda i: (0, i, 0)),
            pl.BlockSpec((_BM, 8), lambda i: (i, 0)),
        ],
        out_shape=[
            jax.ShapeDtypeStruct((NQ, N, QD), _bf16),
            jax.ShapeDtypeStruct((N, 8), _f32),
        ],
    )(ph, wc2)


def _tc3_body(q_ref, out_ref):
    out_ref[...] = _gat_merge(q_ref[...])


def _tc3(q):
    return pl.pallas_call(
        _tc3_body,
        grid=(N // _BM,),
        in_specs=[pl.BlockSpec((NQ, _BM, DW), lambda i: (0, i, 0))],
        out_specs=pl.BlockSpec((_BM, D), lambda i: (i, 0)),
        out_shape=jax.ShapeDtypeStruct((N, D), _f32),
    )(q)


def _perm_cols(w):
    import numpy as _np
    p32 = _np.empty((QD,), _np.int32)
    p32[0::2] = _np.arange(16)
    p32[1::2] = _np.arange(16, 32)
    idx = _np.concatenate([q * QD + p32 for q in range(NQ)])
    return w[:, idx]


def kernel(features, edge_index, W1_0, a1_0, W1_1, a1_1, W2_0, a2_0):
    ws0 = W1_0 @ a1_0[:D, 0]
    wd0 = W1_0 @ a1_0[D:, 0]
    ws1 = W1_1 @ a1_1[:D, 0]
    wd1 = W1_1 @ a1_1[D:, 0]
    zpad = jnp.zeros_like(ws0)
    sdw1 = jnp.stack([ws0, wd0, ws1, wd1, zpad, zpad, zpad, zpad], axis=1)
    wc1 = jnp.concatenate([_perm_cols(W1_0), _perm_cols(W1_1), sdw1],
                          axis=1)

    z0, z1, sd = _tc1(features, wc1)
    eidx = edge_index.reshape(2, 16, NCH, CHUNK)

    ph = _sc_layer1(z0, z1, sd[:, 0], sd[:, 1], sd[:, 2], sd[:, 3], eidx)

    ws2 = W2_0 @ a2_0[:D, 0]
    wd2 = W2_0 @ a2_0[D:, 0]
    zpad2 = jnp.zeros_like(ws2)
    sdw2 = jnp.stack([ws2, wd2] + [zpad2] * 6, axis=1)
    wc2 = jnp.concatenate([_perm_cols(W2_0), sdw2], axis=1)

    z2, sd2 = _tc2(ph, wc2)
    q = _sc_layer2(z2, sd2[:, 0], sd2[:, 1], eidx)
    return _tc3(q)
